# baseline (speedup 1.0000x reference)
import functools

import jax
import jax.numpy as jnp
from jax import lax
from jax.experimental import pallas as pl
from jax.experimental.pallas import tpu as pltpu
from jax.experimental.pallas import tpu_sc as plsc

N = 10000
D = 128
E = 320000

NC, NS, L = 2, 16, 16
NW = NC * NS
N_PAD = 10112
EP = E // NW
CHUNK = 80
SEC = 5
SCH = 25
NR = 10240
RPT = NR // NS
RB = 1264
GRID = N_PAD // RB

_mesh = plsc.VectorSubcoreMesh(core_axis_name="c", subcore_axis_name="s")
_sc_params = pltpu.CompilerParams(needs_layout_passes=False)


@functools.partial(
    pl.kernel,
    out_type=jax.ShapeDtypeStruct((NW, N_PAD), jnp.float32),
    mesh=_mesh,
    compiler_params=_sc_params,
    scratch_types=[
        pltpu.VMEM((SCH, CHUNK), jnp.int32),
        pltpu.VMEM((SCH, CHUNK), jnp.int32),
        pltpu.VMEM((N_PAD,), jnp.float32),
        pltpu.SemaphoreType.DMA,
        pltpu.SemaphoreType.DMA,
    ],
)
def _deg_kernel(ei_hbm, out_hbm, dst_v0, dst_v1, deg_v, dsem0, dsem1):
    w = lax.axis_index("c") * NS + lax.axis_index("s")
    bufs = ((dst_v0, dsem0), (dst_v1, dsem1))

    def ld(k, b):
        return pltpu.make_async_copy(ei_hbm.at[1, k, w], bufs[b][0], bufs[b][1])

    ld(0, 0).start()

    def zero_body(i, carry):
        deg_v[pl.ds(i * L, L)] = jnp.zeros((L,), jnp.float32)
        return carry

    lax.fori_loop(0, N_PAD // L, zero_body, 0)
    ones = jnp.ones((L,), jnp.float32)
    for k in range(SEC):
        b = k % 2
        if k + 1 < SEC:
            ld(k + 1, 1 - b).start()
        ld(k, b).wait()
        dvb = bufs[b][0]

        def row_body(r, carry, dvb=dvb):
            def vec_body(v, c2):
                plsc.addupdate_scatter(
                    deg_v, [dvb[r, pl.ds(v * L, L)]], ones)
                return c2

            return lax.fori_loop(0, CHUNK // L, vec_body, carry)

        lax.fori_loop(0, SCH, row_body, 0)
    pltpu.sync_copy(deg_v, out_hbm.at[w])


@functools.partial(
    pl.kernel,
    out_type=jax.ShapeDtypeStruct((NC, NR, D), jnp.float32),
    mesh=_mesh,
    compiler_params=_sc_params,
    scratch_types=[
        pltpu.VMEM((SCH, CHUNK), jnp.int32),
        pltpu.VMEM((SCH, CHUNK), jnp.int32),
        pltpu.VMEM((2, CHUNK, D), jnp.float32),
        pltpu.VMEM_SHARED((NR, D), jnp.float32),
        pltpu.SemaphoreType.DMA,
        pltpu.SemaphoreType.DMA,
    ],
)
def _scat_kernel(g_hbm, ei_hbm, out_hbm,
                 src_v, dst_v, rows_v, acc_sh, sem0, sem1):
    c = lax.axis_index("c")
    s = lax.axis_index("s")
    w = c * NS + s
    stripe = pl.ds(s * RPT, RPT)
    @pl.when(s < NS - 1)
    def _():
        pltpu.sync_copy(g_hbm.at[stripe], acc_sh.at[stripe])

    @pl.when(s == NS - 1)
    def _():
        last = pl.ds((NS - 1) * RPT, N - (NS - 1) * RPT)
        pltpu.sync_copy(g_hbm.at[last], acc_sh.at[last])

    plsc.subcore_barrier()

    def gat(i, buf, sem):
        return pltpu.make_async_copy(g_hbm.at[src_v.at[i]], rows_v.at[buf], sem)

    npairs = SCH // 2

    def sec_body(k, carry):
        pltpu.sync_copy(ei_hbm.at[0, k, w], src_v)
        pltpu.sync_copy(ei_hbm.at[1, k, w], dst_v)
        gat(0, 0, sem0).start()

        def body(j, carry2):
            i0 = j * 2
            gat(i0 + 1, 1, sem1).start()
            gat(i0, 0, sem0).wait()
            pltpu.sync_copy(rows_v.at[0], acc_sh.at[dst_v.at[i0]], add=True)
            gat(i0 + 2, 0, sem0).start()
            gat(i0 + 1, 1, sem1).wait()
            pltpu.sync_copy(rows_v.at[1], acc_sh.at[dst_v.at[i0 + 1]], add=True)
            return carry2

        lax.fori_loop(0, npairs, body, 0)
        gat(SCH - 1, 0, sem0).wait()
        pltpu.sync_copy(rows_v.at[0], acc_sh.at[dst_v.at[SCH - 1]], add=True)
        return carry

    lax.fori_loop(0, SEC, sec_body, 0)
    plsc.subcore_barrier()
    pltpu.sync_copy(acc_sh.at[stripe], out_hbm.at[c, stripe])


def _dis_body(deg_ref, o_ref):
    o_ref[...] = lax.rsqrt(jnp.sum(deg_ref[...], axis=0) + 1.0)


_dis_call = pl.pallas_call(
    _dis_body,
    out_shape=jax.ShapeDtypeStruct((N_PAD,), jnp.float32),
)


def _mm1_body(x_ref, w_ref, dis_ref, o_ref):
    o_ref[...] = jnp.dot(x_ref[...], w_ref[...],
                         preferred_element_type=jnp.float32) * dis_ref[...]


_mm1_call = pl.pallas_call(
    _mm1_body,
    grid=(GRID,),
    in_specs=[
        pl.BlockSpec((RB, D), lambda i: (i, 0)),
        pl.BlockSpec((D, D), lambda i: (0, 0)),
        pl.BlockSpec((RB, 1), lambda i: (i, 0)),
    ],
    out_specs=pl.BlockSpec((RB, D), lambda i: (i, 0)),
    out_shape=jax.ShapeDtypeStruct((N, D), jnp.float32),
)


def _mm2_body(a_ref, g_ref, dis_ref, b_ref, w_ref, o_ref):
    t = (a_ref[0] + a_ref[1] - g_ref[...]) * dis_ref[...] + b_ref[...]
    t = jnp.maximum(t, 0.0)
    o_ref[...] = jnp.dot(t, w_ref[...],
                         preferred_element_type=jnp.float32) * dis_ref[...]


_mm2_call = pl.pallas_call(
    _mm2_body,
    grid=(GRID,),
    in_specs=[
        pl.BlockSpec((NC, RB, D), lambda i: (0, i, 0)),
        pl.BlockSpec((RB, D), lambda i: (i, 0)),
        pl.BlockSpec((RB, 1), lambda i: (i, 0)),
        pl.BlockSpec((1, D), lambda i: (0, 0)),
        pl.BlockSpec((D, D), lambda i: (0, 0)),
    ],
    out_specs=pl.BlockSpec((RB, D), lambda i: (i, 0)),
    out_shape=jax.ShapeDtypeStruct((N, D), jnp.float32),
)


def _fin_body(a_ref, g_ref, dis_ref, b_ref, o_ref):
    o_ref[...] = (a_ref[0] + a_ref[1] - g_ref[...]) * dis_ref[...] + b_ref[...]


_fin_call = pl.pallas_call(
    _fin_body,
    grid=(GRID,),
    in_specs=[
        pl.BlockSpec((NC, RB, D), lambda i: (0, i, 0)),
        pl.BlockSpec((RB, D), lambda i: (i, 0)),
        pl.BlockSpec((RB, 1), lambda i: (i, 0)),
        pl.BlockSpec((1, D), lambda i: (0, 0)),
    ],
    out_specs=pl.BlockSpec((RB, D), lambda i: (i, 0)),
    out_shape=jax.ShapeDtypeStruct((N, D), jnp.float32),
)


def kernel(x, edge_index, W1, b1, W2, b2):
    ei = edge_index.astype(jnp.int32).reshape(2, SEC, NW, SCH, CHUNK)

    deg_parts = _deg_kernel(ei)
    dis = _dis_call(deg_parts)
    dis_col = dis.reshape(N_PAD, 1)

    g1 = _mm1_call(x, W1, dis_col)
    acc1 = _scat_kernel(g1, ei)
    g2 = _mm2_call(acc1, g1, dis_col, b1.reshape(1, D), W2)
    acc2 = _scat_kernel(g2, ei)
    return _fin_call(acc2, g2, dis_col, b2.reshape(1, D))


# speedup vs baseline: 1.0851x; 1.0851x over previous
import functools

import jax
import jax.numpy as jnp
from jax import lax
from jax.experimental import pallas as pl
from jax.experimental.pallas import tpu as pltpu
from jax.experimental.pallas import tpu_sc as plsc

N = 10000
D = 128
E = 320000

NC, NS, L = 2, 16, 16
NW = NC * NS
N_PAD = 10112
EP = E // NW
CHUNK = 125
SEC = 4
SCH = 20
CHUNK_D = 80
SEC_D = 5
SCH_D = 25
NR = 10240
RPT = NR // NS
RB = 1264
GRID = N_PAD // RB

_mesh = plsc.VectorSubcoreMesh(core_axis_name="c", subcore_axis_name="s")
_sc_params = pltpu.CompilerParams(needs_layout_passes=False)


@functools.partial(
    pl.kernel,
    out_type=jax.ShapeDtypeStruct((NW, N_PAD), jnp.float32),
    mesh=_mesh,
    compiler_params=_sc_params,
    scratch_types=[
        pltpu.VMEM((SCH_D, CHUNK_D), jnp.int32),
        pltpu.VMEM((SCH_D, CHUNK_D), jnp.int32),
        pltpu.VMEM((N_PAD,), jnp.float32),
        pltpu.SemaphoreType.DMA,
        pltpu.SemaphoreType.DMA,
    ],
)
def _deg_kernel(ei_hbm, out_hbm, dst_v0, dst_v1, deg_v, dsem0, dsem1):
    w = lax.axis_index("c") * NS + lax.axis_index("s")
    bufs = ((dst_v0, dsem0), (dst_v1, dsem1))

    def ld(k, b):
        return pltpu.make_async_copy(ei_hbm.at[1, k, w], bufs[b][0], bufs[b][1])

    ld(0, 0).start()

    def zero_body(i, carry):
        deg_v[pl.ds(i * L, L)] = jnp.zeros((L,), jnp.float32)
        return carry

    lax.fori_loop(0, N_PAD // L, zero_body, 0)
    ones = jnp.ones((L,), jnp.float32)
    for k in range(SEC_D):
        b = k % 2
        if k + 1 < SEC_D:
            ld(k + 1, 1 - b).start()
        ld(k, b).wait()
        dvb = bufs[b][0]

        def row_body(r, carry, dvb=dvb):
            def vec_body(v, c2):
                plsc.addupdate_scatter(
                    deg_v, [dvb[r, pl.ds(v * L, L)]], ones)
                return c2

            return lax.fori_loop(0, CHUNK_D // L, vec_body, carry)

        lax.fori_loop(0, SCH_D, row_body, 0)
    pltpu.sync_copy(deg_v, out_hbm.at[w])


@functools.partial(
    pl.kernel,
    out_type=jax.ShapeDtypeStruct((NC, NR, D), jnp.float32),
    mesh=_mesh,
    compiler_params=_sc_params,
    scratch_types=[
        pltpu.VMEM((SCH, CHUNK), jnp.int32),
        pltpu.VMEM((SCH, CHUNK), jnp.int32),
        pltpu.VMEM((2, CHUNK, D), jnp.float32),
        pltpu.VMEM_SHARED((NR, D), jnp.float32),
        pltpu.SemaphoreType.DMA,
        pltpu.SemaphoreType.DMA,
    ],
)
def _scat_kernel(g_hbm, ei_hbm, out_hbm,
                 src_v, dst_v, rows_v, acc_sh, sem0, sem1):
    c = lax.axis_index("c")
    s = lax.axis_index("s")
    w = c * NS + s
    stripe = pl.ds(s * RPT, RPT)
    @pl.when(s < NS - 1)
    def _():
        pltpu.sync_copy(g_hbm.at[stripe], acc_sh.at[stripe])

    @pl.when(s == NS - 1)
    def _():
        last = pl.ds((NS - 1) * RPT, N - (NS - 1) * RPT)
        pltpu.sync_copy(g_hbm.at[last], acc_sh.at[last])

    plsc.subcore_barrier()

    def gat(i, buf, sem):
        return pltpu.make_async_copy(g_hbm.at[src_v.at[i]], rows_v.at[buf], sem)

    npairs = SCH // 2

    def sec_body(k, carry):
        pltpu.sync_copy(ei_hbm.at[0, k, w], src_v)
        pltpu.sync_copy(ei_hbm.at[1, k, w], dst_v)
        gat(0, 0, sem0).start()

        def body(j, carry2):
            i0 = j * 2
            gat(i0 + 1, 1, sem1).start()
            gat(i0, 0, sem0).wait()
            pltpu.sync_copy(rows_v.at[0], acc_sh.at[dst_v.at[i0]], add=True)

            @pl.when(j < npairs - 1)
            def _():
                gat(i0 + 2, 0, sem0).start()

            gat(i0 + 1, 1, sem1).wait()
            pltpu.sync_copy(rows_v.at[1], acc_sh.at[dst_v.at[i0 + 1]], add=True)
            return carry2

        lax.fori_loop(0, npairs, body, 0)
        return carry

    lax.fori_loop(0, SEC, sec_body, 0)
    plsc.subcore_barrier()
    pltpu.sync_copy(acc_sh.at[stripe], out_hbm.at[c, stripe])


def _dis_body(deg_ref, o_ref):
    o_ref[...] = lax.rsqrt(jnp.sum(deg_ref[...], axis=0) + 1.0)


_dis_call = pl.pallas_call(
    _dis_body,
    out_shape=jax.ShapeDtypeStruct((N_PAD,), jnp.float32),
)


def _mm1_body(x_ref, w_ref, dis_ref, o_ref):
    o_ref[...] = jnp.dot(x_ref[...], w_ref[...],
                         preferred_element_type=jnp.float32) * dis_ref[...]


_mm1_call = pl.pallas_call(
    _mm1_body,
    grid=(GRID,),
    in_specs=[
        pl.BlockSpec((RB, D), lambda i: (i, 0)),
        pl.BlockSpec((D, D), lambda i: (0, 0)),
        pl.BlockSpec((RB, 1), lambda i: (i, 0)),
    ],
    out_specs=pl.BlockSpec((RB, D), lambda i: (i, 0)),
    out_shape=jax.ShapeDtypeStruct((N, D), jnp.float32),
)


def _mm2_body(a_ref, g_ref, dis_ref, b_ref, w_ref, o_ref):
    t = (a_ref[0] + a_ref[1] - g_ref[...]) * dis_ref[...] + b_ref[...]
    t = jnp.maximum(t, 0.0)
    o_ref[...] = jnp.dot(t, w_ref[...],
                         preferred_element_type=jnp.float32) * dis_ref[...]


_mm2_call = pl.pallas_call(
    _mm2_body,
    grid=(GRID,),
    in_specs=[
        pl.BlockSpec((NC, RB, D), lambda i: (0, i, 0)),
        pl.BlockSpec((RB, D), lambda i: (i, 0)),
        pl.BlockSpec((RB, 1), lambda i: (i, 0)),
        pl.BlockSpec((1, D), lambda i: (0, 0)),
        pl.BlockSpec((D, D), lambda i: (0, 0)),
    ],
    out_specs=pl.BlockSpec((RB, D), lambda i: (i, 0)),
    out_shape=jax.ShapeDtypeStruct((N, D), jnp.float32),
)


def _fin_body(a_ref, g_ref, dis_ref, b_ref, o_ref):
    o_ref[...] = (a_ref[0] + a_ref[1] - g_ref[...]) * dis_ref[...] + b_ref[...]


_fin_call = pl.pallas_call(
    _fin_body,
    grid=(GRID,),
    in_specs=[
        pl.BlockSpec((NC, RB, D), lambda i: (0, i, 0)),
        pl.BlockSpec((RB, D), lambda i: (i, 0)),
        pl.BlockSpec((RB, 1), lambda i: (i, 0)),
        pl.BlockSpec((1, D), lambda i: (0, 0)),
    ],
    out_specs=pl.BlockSpec((RB, D), lambda i: (i, 0)),
    out_shape=jax.ShapeDtypeStruct((N, D), jnp.float32),
)


def kernel(x, edge_index, W1, b1, W2, b2):
    ei32 = edge_index.astype(jnp.int32)
    ei = ei32.reshape(2, SEC, NW, SCH, CHUNK)
    ei_d = ei32.reshape(2, SEC_D, NW, SCH_D, CHUNK_D)

    deg_parts = _deg_kernel(ei_d)
    dis = _dis_call(deg_parts)
    dis_col = dis.reshape(N_PAD, 1)

    g1 = _mm1_call(x, W1, dis_col)
    acc1 = _scat_kernel(g1, ei)
    g2 = _mm2_call(acc1, g1, dis_col, b1.reshape(1, D), W2)
    acc2 = _scat_kernel(g2, ei)
    return _fin_call(acc2, g2, dis_col, b2.reshape(1, D))


# speedup vs baseline: 1.0982x; 1.0120x over previous
import functools

import jax
import jax.numpy as jnp
from jax import lax
from jax.experimental import pallas as pl
from jax.experimental.pallas import tpu as pltpu
from jax.experimental.pallas import tpu_sc as plsc

N = 10000
D = 128
E = 320000

NC, NS, L = 2, 16, 16
NW = NC * NS
N_PAD = 10112
EP = E // NW
CHUNK = 125
SEC = 4
SCH = 20
CHUNK_D = 80
SEC_D = 5
SCH_D = 25
NR = 10240
RPT = NR // NS
RB = 1264
GRID = N_PAD // RB

_mesh = plsc.VectorSubcoreMesh(core_axis_name="c", subcore_axis_name="s")
_sc_params = pltpu.CompilerParams(needs_layout_passes=False)


@functools.partial(
    pl.kernel,
    out_type=jax.ShapeDtypeStruct((NW, N_PAD), jnp.float32),
    mesh=_mesh,
    compiler_params=_sc_params,
    scratch_types=[
        pltpu.VMEM((SCH_D, CHUNK_D), jnp.int32),
        pltpu.VMEM((SCH_D, CHUNK_D), jnp.int32),
        pltpu.VMEM((N_PAD,), jnp.float32),
        pltpu.SemaphoreType.DMA,
        pltpu.SemaphoreType.DMA,
    ],
)
def _deg_kernel(ei_hbm, out_hbm, dst_v0, dst_v1, deg_v, dsem0, dsem1):
    w = lax.axis_index("c") * NS + lax.axis_index("s")
    bufs = ((dst_v0, dsem0), (dst_v1, dsem1))

    def ld(k, b):
        return pltpu.make_async_copy(ei_hbm.at[1, k, w], bufs[b][0], bufs[b][1])

    ld(0, 0).start()

    def zero_body(i, carry):
        deg_v[pl.ds(i * L, L)] = jnp.zeros((L,), jnp.float32)
        return carry

    lax.fori_loop(0, N_PAD // L, zero_body, 0)
    ones = jnp.ones((L,), jnp.float32)
    for k in range(SEC_D):
        b = k % 2
        if k + 1 < SEC_D:
            ld(k + 1, 1 - b).start()
        ld(k, b).wait()
        dvb = bufs[b][0]

        def row_body(r, carry, dvb=dvb):
            def vec_body(v, c2):
                plsc.addupdate_scatter(
                    deg_v, [dvb[r, pl.ds(v * L, L)]], ones)
                return c2

            return lax.fori_loop(0, CHUNK_D // L, vec_body, carry)

        lax.fori_loop(0, SCH_D, row_body, 0)
    pltpu.sync_copy(deg_v, out_hbm.at[w])


@functools.partial(
    pl.kernel,
    out_type=jax.ShapeDtypeStruct((NC, NR, D), jnp.float32),
    mesh=_mesh,
    compiler_params=_sc_params,
    scratch_types=[
        pltpu.VMEM((SCH, CHUNK), jnp.int32),
        pltpu.VMEM((SCH, CHUNK), jnp.int32),
        pltpu.VMEM((2, CHUNK, D), jnp.float32),
        pltpu.VMEM_SHARED((NR, D), jnp.float32),
        pltpu.SemaphoreType.DMA,
        pltpu.SemaphoreType.DMA,
    ],
)
def _scat_kernel(g_hbm, ei_hbm, out_hbm,
                 src_v, dst_v, rows_v, acc_sh, sem0, sem1):
    c = lax.axis_index("c")
    s = lax.axis_index("s")
    w = c * NS + s
    stripe = pl.ds(s * RPT, RPT)
    @pl.when(s < NS - 1)
    def _():
        pltpu.sync_copy(g_hbm.at[stripe], acc_sh.at[stripe])

    @pl.when(s == NS - 1)
    def _():
        last = pl.ds((NS - 1) * RPT, N - (NS - 1) * RPT)
        pltpu.sync_copy(g_hbm.at[last], acc_sh.at[last])

    plsc.subcore_barrier()

    def gat(i, buf, sem):
        return pltpu.make_async_copy(g_hbm.at[src_v.at[i]], rows_v.at[buf], sem)

    npairs = SCH // 2

    def sec_body(k, carry):
        pltpu.sync_copy(ei_hbm.at[0, k, w], src_v)
        pltpu.sync_copy(ei_hbm.at[1, k, w], dst_v)
        gat(0, 0, sem0).start()

        def body(j, carry2):
            i0 = j * 2
            gat(i0 + 1, 1, sem1).start()
            gat(i0, 0, sem0).wait()
            pltpu.sync_copy(rows_v.at[0], acc_sh.at[dst_v.at[i0]], add=True)

            @pl.when(j < npairs - 1)
            def _():
                gat(i0 + 2, 0, sem0).start()

            gat(i0 + 1, 1, sem1).wait()
            pltpu.sync_copy(rows_v.at[1], acc_sh.at[dst_v.at[i0 + 1]], add=True)
            return carry2

        lax.fori_loop(0, npairs, body, 0)
        return carry

    lax.fori_loop(0, SEC, sec_body, 0)
    plsc.subcore_barrier()
    pltpu.sync_copy(acc_sh.at[stripe], out_hbm.at[c, stripe])


def _dis_body(deg_ref, o_ref):
    d = lax.rsqrt(jnp.sum(deg_ref[...], axis=0) + 1.0)
    o_ref[...] = d.reshape(N_PAD, 1)


_dis_call = pl.pallas_call(
    _dis_body,
    out_shape=jax.ShapeDtypeStruct((N_PAD, 1), jnp.float32),
)


def _mm1_body(x_ref, w_ref, dis_ref, o_ref):
    o_ref[...] = jnp.dot(x_ref[...], w_ref[...],
                         preferred_element_type=jnp.float32) * dis_ref[...]


_mm1_call = pl.pallas_call(
    _mm1_body,
    grid=(GRID,),
    in_specs=[
        pl.BlockSpec((RB, D), lambda i: (i, 0)),
        pl.BlockSpec((D, D), lambda i: (0, 0)),
        pl.BlockSpec((RB, 1), lambda i: (i, 0)),
    ],
    out_specs=pl.BlockSpec((RB, D), lambda i: (i, 0)),
    out_shape=jax.ShapeDtypeStruct((N, D), jnp.float32),
)


def _mm2_body(a_ref, g_ref, dis_ref, b_ref, w_ref, o_ref):
    t = (a_ref[0] + a_ref[1] - g_ref[...]) * dis_ref[...] + b_ref[...]
    t = jnp.maximum(t, 0.0)
    o_ref[...] = jnp.dot(t, w_ref[...],
                         preferred_element_type=jnp.float32) * dis_ref[...]


_mm2_call = pl.pallas_call(
    _mm2_body,
    grid=(GRID,),
    in_specs=[
        pl.BlockSpec((NC, RB, D), lambda i: (0, i, 0)),
        pl.BlockSpec((RB, D), lambda i: (i, 0)),
        pl.BlockSpec((RB, 1), lambda i: (i, 0)),
        pl.BlockSpec((1, D), lambda i: (0, 0)),
        pl.BlockSpec((D, D), lambda i: (0, 0)),
    ],
    out_specs=pl.BlockSpec((RB, D), lambda i: (i, 0)),
    out_shape=jax.ShapeDtypeStruct((N, D), jnp.float32),
)


def _fin_body(a_ref, g_ref, dis_ref, b_ref, o_ref):
    o_ref[...] = (a_ref[0] + a_ref[1] - g_ref[...]) * dis_ref[...] + b_ref[...]


_fin_call = pl.pallas_call(
    _fin_body,
    grid=(GRID,),
    in_specs=[
        pl.BlockSpec((NC, RB, D), lambda i: (0, i, 0)),
        pl.BlockSpec((RB, D), lambda i: (i, 0)),
        pl.BlockSpec((RB, 1), lambda i: (i, 0)),
        pl.BlockSpec((1, D), lambda i: (0, 0)),
    ],
    out_specs=pl.BlockSpec((RB, D), lambda i: (i, 0)),
    out_shape=jax.ShapeDtypeStruct((N, D), jnp.float32),
)


def kernel(x, edge_index, W1, b1, W2, b2):
    ei32 = edge_index.astype(jnp.int32)
    ei = ei32.reshape(2, SEC, NW, SCH, CHUNK)
    ei_d = ei32.reshape(2, SEC_D, NW, SCH_D, CHUNK_D)

    deg_parts = _deg_kernel(ei_d)
    dis_col = _dis_call(deg_parts)

    g1 = _mm1_call(x, W1, dis_col)
    acc1 = _scat_kernel(g1, ei)
    g2 = _mm2_call(acc1, g1, dis_col, b1.reshape(1, D), W2)
    acc2 = _scat_kernel(g2, ei)
    return _fin_call(acc2, g2, dis_col, b2.reshape(1, D))


# speedup vs baseline: 1.1130x; 1.0135x over previous
import functools

import jax
import jax.numpy as jnp
from jax import lax
from jax.experimental import pallas as pl
from jax.experimental.pallas import tpu as pltpu
from jax.experimental.pallas import tpu_sc as plsc

N = 10000
D = 128
E = 320000

NC, NS, L = 2, 16, 16
NW = NC * NS
N_PAD = 10112
EP = E // NW
CHUNK = 125
SEC = 4
SCH = 20
CHUNK_D = 80
SEC_D = 5
SCH_D = 25
NR = 10240
RPT = NR // NS
RB = 2528
GRID = N_PAD // RB

_mesh = plsc.VectorSubcoreMesh(core_axis_name="c", subcore_axis_name="s")
_sc_params = pltpu.CompilerParams(needs_layout_passes=False)


@functools.partial(
    pl.kernel,
    out_type=jax.ShapeDtypeStruct((NW, N_PAD), jnp.float32),
    mesh=_mesh,
    compiler_params=_sc_params,
    scratch_types=[
        pltpu.VMEM((SCH_D, CHUNK_D), jnp.int32),
        pltpu.VMEM((SCH_D, CHUNK_D), jnp.int32),
        pltpu.VMEM((N_PAD,), jnp.float32),
        pltpu.SemaphoreType.DMA,
        pltpu.SemaphoreType.DMA,
    ],
)
def _deg_kernel(ei_hbm, out_hbm, dst_v0, dst_v1, deg_v, dsem0, dsem1):
    w = lax.axis_index("c") * NS + lax.axis_index("s")
    bufs = ((dst_v0, dsem0), (dst_v1, dsem1))

    def ld(k, b):
        return pltpu.make_async_copy(ei_hbm.at[1, k, w], bufs[b][0], bufs[b][1])

    ld(0, 0).start()

    def zero_body(i, carry):
        deg_v[pl.ds(i * L, L)] = jnp.zeros((L,), jnp.float32)
        return carry

    lax.fori_loop(0, N_PAD // L, zero_body, 0)
    ones = jnp.ones((L,), jnp.float32)
    for k in range(SEC_D):
        b = k % 2
        if k + 1 < SEC_D:
            ld(k + 1, 1 - b).start()
        ld(k, b).wait()
        dvb = bufs[b][0]

        def row_body(r, carry, dvb=dvb):
            def vec_body(v, c2):
                plsc.addupdate_scatter(
                    deg_v, [dvb[r, pl.ds(v * L, L)]], ones)
                return c2

            return lax.fori_loop(0, CHUNK_D // L, vec_body, carry)

        lax.fori_loop(0, SCH_D, row_body, 0)
    pltpu.sync_copy(deg_v, out_hbm.at[w])


@functools.partial(
    pl.kernel,
    out_type=jax.ShapeDtypeStruct((NC, NR, D), jnp.float32),
    mesh=_mesh,
    compiler_params=_sc_params,
    scratch_types=[
        pltpu.VMEM((SCH, CHUNK), jnp.int32),
        pltpu.VMEM((SCH, CHUNK), jnp.int32),
        pltpu.VMEM((2, CHUNK, D), jnp.float32),
        pltpu.VMEM_SHARED((NR, D), jnp.float32),
        pltpu.SemaphoreType.DMA,
        pltpu.SemaphoreType.DMA,
    ],
)
def _scat_kernel(g_hbm, ei_hbm, out_hbm,
                 src_v, dst_v, rows_v, acc_sh, sem0, sem1):
    c = lax.axis_index("c")
    s = lax.axis_index("s")
    w = c * NS + s
    stripe = pl.ds(s * RPT, RPT)
    @pl.when(s < NS - 1)
    def _():
        pltpu.sync_copy(g_hbm.at[stripe], acc_sh.at[stripe])

    @pl.when(s == NS - 1)
    def _():
        last = pl.ds((NS - 1) * RPT, N - (NS - 1) * RPT)
        pltpu.sync_copy(g_hbm.at[last], acc_sh.at[last])

    plsc.subcore_barrier()

    def gat(i, buf, sem):
        return pltpu.make_async_copy(g_hbm.at[src_v.at[i]], rows_v.at[buf], sem)

    npairs = SCH // 2

    def sec_body(k, carry):
        pltpu.sync_copy(ei_hbm.at[0, k, w], src_v)
        pltpu.sync_copy(ei_hbm.at[1, k, w], dst_v)
        gat(0, 0, sem0).start()

        def body(j, carry2):
            i0 = j * 2
            gat(i0 + 1, 1, sem1).start()
            gat(i0, 0, sem0).wait()
            pltpu.sync_copy(rows_v.at[0], acc_sh.at[dst_v.at[i0]], add=True)

            @pl.when(j < npairs - 1)
            def _():
                gat(i0 + 2, 0, sem0).start()

            gat(i0 + 1, 1, sem1).wait()
            pltpu.sync_copy(rows_v.at[1], acc_sh.at[dst_v.at[i0 + 1]], add=True)
            return carry2

        lax.fori_loop(0, npairs, body, 0)
        return carry

    lax.fori_loop(0, SEC, sec_body, 0)
    plsc.subcore_barrier()
    pltpu.sync_copy(acc_sh.at[stripe], out_hbm.at[c, stripe])


def _dis_body(deg_ref, o_ref):
    d = lax.rsqrt(jnp.sum(deg_ref[...], axis=0) + 1.0)
    o_ref[...] = d.reshape(N_PAD, 1)


_dis_call = pl.pallas_call(
    _dis_body,
    out_shape=jax.ShapeDtypeStruct((N_PAD, 1), jnp.float32),
)


def _mm1_body(x_ref, w_ref, dis_ref, o_ref):
    o_ref[...] = jnp.dot(x_ref[...], w_ref[...],
                         preferred_element_type=jnp.float32) * dis_ref[...]


_mm1_call = pl.pallas_call(
    _mm1_body,
    grid=(GRID,),
    in_specs=[
        pl.BlockSpec((RB, D), lambda i: (i, 0)),
        pl.BlockSpec((D, D), lambda i: (0, 0)),
        pl.BlockSpec((RB, 1), lambda i: (i, 0)),
    ],
    out_specs=pl.BlockSpec((RB, D), lambda i: (i, 0)),
    out_shape=jax.ShapeDtypeStruct((N, D), jnp.float32),
)


def _mm2_body(a_ref, g_ref, dis_ref, b_ref, w_ref, o_ref):
    t = (a_ref[0] + a_ref[1] - g_ref[...]) * dis_ref[...] + b_ref[...]
    t = jnp.maximum(t, 0.0)
    o_ref[...] = jnp.dot(t, w_ref[...],
                         preferred_element_type=jnp.float32) * dis_ref[...]


_mm2_call = pl.pallas_call(
    _mm2_body,
    grid=(GRID,),
    in_specs=[
        pl.BlockSpec((NC, RB, D), lambda i: (0, i, 0)),
        pl.BlockSpec((RB, D), lambda i: (i, 0)),
        pl.BlockSpec((RB, 1), lambda i: (i, 0)),
        pl.BlockSpec((1, D), lambda i: (0, 0)),
        pl.BlockSpec((D, D), lambda i: (0, 0)),
    ],
    out_specs=pl.BlockSpec((RB, D), lambda i: (i, 0)),
    out_shape=jax.ShapeDtypeStruct((N, D), jnp.float32),
)


def _fin_body(a_ref, g_ref, dis_ref, b_ref, o_ref):
    o_ref[...] = (a_ref[0] + a_ref[1] - g_ref[...]) * dis_ref[...] + b_ref[...]


_fin_call = pl.pallas_call(
    _fin_body,
    grid=(GRID,),
    in_specs=[
        pl.BlockSpec((NC, RB, D), lambda i: (0, i, 0)),
        pl.BlockSpec((RB, D), lambda i: (i, 0)),
        pl.BlockSpec((RB, 1), lambda i: (i, 0)),
        pl.BlockSpec((1, D), lambda i: (0, 0)),
    ],
    out_specs=pl.BlockSpec((RB, D), lambda i: (i, 0)),
    out_shape=jax.ShapeDtypeStruct((N, D), jnp.float32),
)


def kernel(x, edge_index, W1, b1, W2, b2):
    ei32 = edge_index.astype(jnp.int32)
    ei = ei32.reshape(2, SEC, NW, SCH, CHUNK)
    ei_d = ei32.reshape(2, SEC_D, NW, SCH_D, CHUNK_D)

    deg_parts = _deg_kernel(ei_d)
    dis_col = _dis_call(deg_parts)

    g1 = _mm1_call(x, W1, dis_col)
    acc1 = _scat_kernel(g1, ei)
    g2 = _mm2_call(acc1, g1, dis_col, b1.reshape(1, D), W2)
    acc2 = _scat_kernel(g2, ei)
    return _fin_call(acc2, g2, dis_col, b2.reshape(1, D))


# speedup vs baseline: 1.1184x; 1.0048x over previous
import functools

import jax
import jax.numpy as jnp
from jax import lax
from jax.experimental import pallas as pl
from jax.experimental.pallas import tpu as pltpu
from jax.experimental.pallas import tpu_sc as plsc

N = 10000
D = 128
E = 320000

NC, NS, L = 2, 16, 16
NW = NC * NS
N_PAD = 10112
EP = E // NW
CHUNK = 125
SEC = 4
SCH = 20
CHUNK_D = 80
SEC_D = 5
SCH_D = 25
NR = 10240
RPT = NR // NS
RB = 5056
GRID = N_PAD // RB

_mesh = plsc.VectorSubcoreMesh(core_axis_name="c", subcore_axis_name="s")
_sc_params = pltpu.CompilerParams(needs_layout_passes=False)


@functools.partial(
    pl.kernel,
    out_type=jax.ShapeDtypeStruct((NW, N_PAD), jnp.float32),
    mesh=_mesh,
    compiler_params=_sc_params,
    scratch_types=[
        pltpu.VMEM((SCH_D, CHUNK_D), jnp.int32),
        pltpu.VMEM((SCH_D, CHUNK_D), jnp.int32),
        pltpu.VMEM((N_PAD,), jnp.float32),
        pltpu.SemaphoreType.DMA,
        pltpu.SemaphoreType.DMA,
    ],
)
def _deg_kernel(ei_hbm, out_hbm, dst_v0, dst_v1, deg_v, dsem0, dsem1):
    w = lax.axis_index("c") * NS + lax.axis_index("s")
    bufs = ((dst_v0, dsem0), (dst_v1, dsem1))

    def ld(k, b):
        return pltpu.make_async_copy(ei_hbm.at[1, k, w], bufs[b][0], bufs[b][1])

    ld(0, 0).start()

    def zero_body(i, carry):
        deg_v[pl.ds(i * L, L)] = jnp.zeros((L,), jnp.float32)
        return carry

    lax.fori_loop(0, N_PAD // L, zero_body, 0)
    ones = jnp.ones((L,), jnp.float32)
    for k in range(SEC_D):
        b = k % 2
        if k + 1 < SEC_D:
            ld(k + 1, 1 - b).start()
        ld(k, b).wait()
        dvb = bufs[b][0]

        def row_body(r, carry, dvb=dvb):
            def vec_body(v, c2):
                plsc.addupdate_scatter(
                    deg_v, [dvb[r, pl.ds(v * L, L)]], ones)
                return c2

            return lax.fori_loop(0, CHUNK_D // L, vec_body, carry)

        lax.fori_loop(0, SCH_D, row_body, 0)
    pltpu.sync_copy(deg_v, out_hbm.at[w])


@functools.partial(
    pl.kernel,
    out_type=jax.ShapeDtypeStruct((NC, NR, D), jnp.float32),
    mesh=_mesh,
    compiler_params=_sc_params,
    scratch_types=[
        pltpu.VMEM((SCH, CHUNK), jnp.int32),
        pltpu.VMEM((SCH, CHUNK), jnp.int32),
        pltpu.VMEM((2, CHUNK, D), jnp.float32),
        pltpu.VMEM_SHARED((NR, D), jnp.float32),
        pltpu.SemaphoreType.DMA,
        pltpu.SemaphoreType.DMA,
    ],
)
def _scat_kernel(g_hbm, ei_hbm, out_hbm,
                 src_v, dst_v, rows_v, acc_sh, sem0, sem1):
    c = lax.axis_index("c")
    s = lax.axis_index("s")
    w = c * NS + s
    stripe = pl.ds(s * RPT, RPT)
    @pl.when(s < NS - 1)
    def _():
        pltpu.sync_copy(g_hbm.at[stripe], acc_sh.at[stripe])

    @pl.when(s == NS - 1)
    def _():
        last = pl.ds((NS - 1) * RPT, N - (NS - 1) * RPT)
        pltpu.sync_copy(g_hbm.at[last], acc_sh.at[last])

    plsc.subcore_barrier()

    def gat(i, buf, sem):
        return pltpu.make_async_copy(g_hbm.at[src_v.at[i]], rows_v.at[buf], sem)

    npairs = SCH // 2

    def sec_body(k, carry):
        pltpu.sync_copy(ei_hbm.at[0, k, w], src_v)
        pltpu.sync_copy(ei_hbm.at[1, k, w], dst_v)
        gat(0, 0, sem0).start()

        def body(j, carry2):
            i0 = j * 2
            gat(i0 + 1, 1, sem1).start()
            gat(i0, 0, sem0).wait()
            pltpu.sync_copy(rows_v.at[0], acc_sh.at[dst_v.at[i0]], add=True)

            @pl.when(j < npairs - 1)
            def _():
                gat(i0 + 2, 0, sem0).start()

            gat(i0 + 1, 1, sem1).wait()
            pltpu.sync_copy(rows_v.at[1], acc_sh.at[dst_v.at[i0 + 1]], add=True)
            return carry2

        lax.fori_loop(0, npairs, body, 0)
        return carry

    lax.fori_loop(0, SEC, sec_body, 0)
    plsc.subcore_barrier()
    pltpu.sync_copy(acc_sh.at[stripe], out_hbm.at[c, stripe])


def _dis_body(deg_ref, o_ref):
    d = lax.rsqrt(jnp.sum(deg_ref[...], axis=0) + 1.0)
    o_ref[...] = d.reshape(N_PAD, 1)


_dis_call = pl.pallas_call(
    _dis_body,
    out_shape=jax.ShapeDtypeStruct((N_PAD, 1), jnp.float32),
)


def _mm1_body(x_ref, w_ref, dis_ref, o_ref):
    o_ref[...] = jnp.dot(x_ref[...], w_ref[...],
                         preferred_element_type=jnp.float32) * dis_ref[...]


_mm1_call = pl.pallas_call(
    _mm1_body,
    grid=(GRID,),
    in_specs=[
        pl.BlockSpec((RB, D), lambda i: (i, 0)),
        pl.BlockSpec((D, D), lambda i: (0, 0)),
        pl.BlockSpec((RB, 1), lambda i: (i, 0)),
    ],
    out_specs=pl.BlockSpec((RB, D), lambda i: (i, 0)),
    out_shape=jax.ShapeDtypeStruct((N, D), jnp.float32),
)


def _mm2_body(a_ref, g_ref, dis_ref, b_ref, w_ref, o_ref):
    t = (a_ref[0] + a_ref[1] - g_ref[...]) * dis_ref[...] + b_ref[...]
    t = jnp.maximum(t, 0.0)
    o_ref[...] = jnp.dot(t, w_ref[...],
                         preferred_element_type=jnp.float32) * dis_ref[...]


_mm2_call = pl.pallas_call(
    _mm2_body,
    grid=(GRID,),
    in_specs=[
        pl.BlockSpec((NC, RB, D), lambda i: (0, i, 0)),
        pl.BlockSpec((RB, D), lambda i: (i, 0)),
        pl.BlockSpec((RB, 1), lambda i: (i, 0)),
        pl.BlockSpec((1, D), lambda i: (0, 0)),
        pl.BlockSpec((D, D), lambda i: (0, 0)),
    ],
    out_specs=pl.BlockSpec((RB, D), lambda i: (i, 0)),
    out_shape=jax.ShapeDtypeStruct((N, D), jnp.float32),
)


def _fin_body(a_ref, g_ref, dis_ref, b_ref, o_ref):
    o_ref[...] = (a_ref[0] + a_ref[1] - g_ref[...]) * dis_ref[...] + b_ref[...]


_fin_call = pl.pallas_call(
    _fin_body,
    grid=(GRID,),
    in_specs=[
        pl.BlockSpec((NC, RB, D), lambda i: (0, i, 0)),
        pl.BlockSpec((RB, D), lambda i: (i, 0)),
        pl.BlockSpec((RB, 1), lambda i: (i, 0)),
        pl.BlockSpec((1, D), lambda i: (0, 0)),
    ],
    out_specs=pl.BlockSpec((RB, D), lambda i: (i, 0)),
    out_shape=jax.ShapeDtypeStruct((N, D), jnp.float32),
)


def kernel(x, edge_index, W1, b1, W2, b2):
    ei32 = edge_index.astype(jnp.int32)
    ei = ei32.reshape(2, SEC, NW, SCH, CHUNK)
    ei_d = ei32.reshape(2, SEC_D, NW, SCH_D, CHUNK_D)

    deg_parts = _deg_kernel(ei_d)
    dis_col = _dis_call(deg_parts)

    g1 = _mm1_call(x, W1, dis_col)
    acc1 = _scat_kernel(g1, ei)
    g2 = _mm2_call(acc1, g1, dis_col, b1.reshape(1, D), W2)
    acc2 = _scat_kernel(g2, ei)
    return _fin_call(acc2, g2, dis_col, b2.reshape(1, D))
